# baseline (speedup 1.0000x reference)
import functools

import jax
import jax.numpy as jnp
from jax import lax
from jax.experimental import pallas as pl
from jax.experimental.pallas import tpu as pltpu
from jax.experimental.pallas import tpu_sc as plsc

VOCAB = 100000
EMBED_DIM = 64
BATCH = 4096

_info = plsc.get_sparse_core_info()
_NC = _info.num_cores
_NS = _info.num_subcores
_L = _info.num_lanes
_NW = _NC * _NS
_BPW = BATCH // _NW
_CHUNKS = EMBED_DIM // _L

_mesh = plsc.VectorSubcoreMesh(core_axis_name="c", subcore_axis_name="s")


@functools.partial(
    pl.kernel,
    mesh=_mesh,
    out_type=jax.ShapeDtypeStruct((_NW, _L), jnp.float32),
    compiler_params=pltpu.CompilerParams(use_tc_tiling_on_sc=False),
    scratch_types=[
        pltpu.VMEM((_BPW,), jnp.int32),
        pltpu.VMEM((_BPW,), jnp.int32),
        pltpu.VMEM((_BPW, EMBED_DIM), jnp.float32),
        pltpu.VMEM((_BPW, EMBED_DIM), jnp.float32),
        pltpu.VMEM((_L,), jnp.float32),
        pltpu.SemaphoreType.DMA,
        pltpu.SemaphoreType.DMA,
    ],
)
def _sc_partial_sumsq(table_hbm, head_hbm, tail_hbm, out_hbm,
                      hidx_v, tidx_v, hrow_v, trow_v, acc_v, sem_h, sem_t):
    wid = lax.axis_index("s") * _NC + lax.axis_index("c")
    base = wid * _BPW
    pltpu.sync_copy(head_hbm.at[pl.ds(base, _BPW)], hidx_v)
    pltpu.sync_copy(tail_hbm.at[pl.ds(base, _BPW)], tidx_v)
    ch = pltpu.async_copy(table_hbm.at[hidx_v], hrow_v, sem_h)
    ct = pltpu.async_copy(table_hbm.at[tidx_v], trow_v, sem_t)
    ch.wait()
    ct.wait()

    def body(r, accs):
        new = []
        for c in range(_CHUNKS):
            h = hrow_v[r, pl.ds(c * _L, _L)]
            t = trow_v[r, pl.ds(c * _L, _L)]
            d = h - t
            new.append(accs[c] + d * d)
        return tuple(new)

    zero = jnp.zeros((_L,), jnp.float32)
    accs = lax.fori_loop(0, _BPW, body, (zero,) * _CHUNKS)
    total = accs[0]
    for c in range(1, _CHUNKS):
        total = total + accs[c]
    acc_v[...] = total
    pltpu.sync_copy(acc_v, out_hbm.at[wid])


def _tc_finish_body(p_ref, o_ref):
    o_ref[...] = jnp.broadcast_to(jnp.sqrt(jnp.sum(p_ref[...])), (1, 1))


def _tc_finish(partials):
    return pl.pallas_call(
        _tc_finish_body,
        out_shape=jax.ShapeDtypeStruct((1, 1), jnp.float32),
    )(partials)


def kernel(table, head, tail):
    partials = _sc_partial_sumsq(
        table, head.astype(jnp.int32), tail.astype(jnp.int32))
    return _tc_finish(partials)[0, 0]
